# baseline (speedup 1.0000x reference)
import jax
import jax.numpy as jnp
from jax.experimental import pallas as pl
from jax.experimental.pallas import tpu as pltpu

DECAY = 0.98
LR = 0.05
B, T, D = 4, 128, 64
M = 65536
BM = 2048
QT = B * T


def _flash_body(hf_ref, sur_ref, mem_ref, str_ref, out_ref,
                qn_ref, acc_ref, dmem_ref, sstr_ref):
    j = pl.program_id(0)
    nb = pl.num_programs(0)

    @pl.when(j == 0)
    def _prologue():
        hfv = hf_ref[...]
        qss = jnp.sum(hfv * hfv, axis=1, keepdims=True)
        qn_ref[...] = (hfv / jnp.maximum(jnp.sqrt(qss), 1e-12)
                       ).astype(jnp.bfloat16)

        s2 = jnp.mean(sur_ref[...], axis=0, keepdims=True)
        r_io = jax.lax.broadcasted_iota(jnp.int32, (T, T), 0)
        c_io = jax.lax.broadcasted_iota(jnp.int32, (T, T), 1)
        eye = (r_io == c_io).astype(jnp.float32)
        s_bc = jnp.broadcast_to(s2, (T, T))
        s_col = jnp.sum(s_bc * eye, axis=1, keepdims=True)
        gt = (s_col > s2).astype(jnp.int32)
        tie = ((s_col == s2) & (r_io < c_io)).astype(jnp.int32)
        rank = jnp.sum(gt + tie, axis=0, keepdims=True)
        ohot = (jnp.broadcast_to(rank, (T, T)) == r_io).astype(jnp.float32)

        mh = (hfv[0:T] + hfv[T:2 * T] + hfv[2 * T:3 * T] + hfv[3 * T:4 * T]) * 0.25
        mss = jnp.sum(mh * mh, axis=1, keepdims=True)
        mhn = mh / jnp.maximum(jnp.sqrt(mss), 1e-12)
        delta = LR * jax.lax.dot_general(
            ohot, mhn, (((1,), (0,)), ((), ())),
            preferred_element_type=jnp.float32)
        dmem_ref[...] = jnp.concatenate(
            [delta, jnp.zeros((BM - T, D), jnp.float32)], axis=0)

        ss_col = jnp.sum(ohot * s_bc, axis=1, keepdims=True)
        sstr_ref[...] = jnp.concatenate(
            [ss_col, jnp.zeros((BM - T, 1), jnp.float32)], axis=0)

        acc_ref[...] = jnp.zeros((QT, D + 1), jnp.float32)

    is0 = jnp.where(j == 0, 1.0, 0.0)
    dec = mem_ref[...] * DECAY + is0 * dmem_ref[...]
    str2 = str_ref[...] * DECAY + is0 * sstr_ref[...]
    nss = jnp.sum(dec * dec, axis=1, keepdims=True)
    rowmul = str2 / jnp.maximum(jnp.sqrt(nss), 1e-12)
    mn = (dec * rowmul).astype(jnp.bfloat16)
    logits = jax.lax.dot_general(
        qn_ref[...], mn, (((1,), (1,)), ((), ())),
        preferred_element_type=jnp.float32)
    p = jnp.exp(logits).astype(jnp.bfloat16)
    dec_aug = jnp.concatenate(
        [dec, jnp.ones((BM, 1), jnp.float32)], axis=1).astype(jnp.bfloat16)
    acc_ref[...] += jax.lax.dot_general(
        p, dec_aug, (((1,), (0,)), ((), ())),
        preferred_element_type=jnp.float32)

    @pl.when(j == nb - 1)
    def _finalize():
        acc = acc_ref[...]
        out_ref[...] = acc[:, :D] / acc[:, D:D + 1]


def kernel(h, surprise, mem, strength):
    hf = h.reshape(QT, D)
    strc = strength.reshape(M, 1)
    out = pl.pallas_call(
        _flash_body,
        grid=(M // BM,),
        in_specs=[
            pl.BlockSpec((QT, D), lambda j: (0, 0)),
            pl.BlockSpec((B, T), lambda j: (0, 0)),
            pl.BlockSpec((BM, D), lambda j: (j, 0)),
            pl.BlockSpec((BM, 1), lambda j: (j, 0)),
        ],
        out_specs=pl.BlockSpec((QT, D), lambda j: (0, 0)),
        out_shape=jax.ShapeDtypeStruct((QT, D), jnp.float32),
        scratch_shapes=[
            pltpu.VMEM((QT, D), jnp.bfloat16),
            pltpu.VMEM((QT, D + 1), jnp.float32),
            pltpu.VMEM((BM, D), jnp.float32),
            pltpu.VMEM((BM, 1), jnp.float32),
        ],
        compiler_params=pltpu.CompilerParams(
            dimension_semantics=("arbitrary",)),
    )(hf, surprise, mem, strc)
    return out.reshape(B, T, D)


# speedup vs baseline: 1.1382x; 1.1382x over previous
import jax
import jax.numpy as jnp
from jax.experimental import pallas as pl
from jax.experimental.pallas import tpu as pltpu

DECAY = 0.98
LR = 0.05
B, T, D = 4, 128, 64
M = 65536
BM = 2048
QT = B * T


def _flash_body(hf_ref, sur_ref, mem_ref, str_ref, out_ref,
                qn_ref, acc_ref, l_ref, dmem_ref, sstr_ref):
    j = pl.program_id(0)
    nb = pl.num_programs(0)
    ones_row = jnp.ones((1, D), jnp.float32)

    @pl.when(j == 0)
    def _prologue():
        hfv = hf_ref[...]
        qss = jnp.sum(hfv * hfv, axis=1, keepdims=True)
        qn_ref[...] = (hfv / jnp.maximum(jnp.sqrt(qss), 1e-12)
                       ).astype(jnp.bfloat16)

        s2 = jnp.mean(sur_ref[...], axis=0, keepdims=True)
        r_io = jax.lax.broadcasted_iota(jnp.int32, (T, T), 0)
        c_io = jax.lax.broadcasted_iota(jnp.int32, (T, T), 1)
        eye = (r_io == c_io).astype(jnp.float32)
        s_bc = jnp.broadcast_to(s2, (T, T))
        s_col = jnp.sum(s_bc * eye, axis=1, keepdims=True)
        gt = (s_col > s2).astype(jnp.int32)
        tie = ((s_col == s2) & (r_io < c_io)).astype(jnp.int32)
        rank = jnp.sum(gt + tie, axis=0, keepdims=True)
        ohot = (jnp.broadcast_to(rank, (T, T)) == r_io).astype(jnp.float32)

        mh = (hfv[0:T] + hfv[T:2 * T] + hfv[2 * T:3 * T] + hfv[3 * T:4 * T]) * 0.25
        mss = jnp.sum(mh * mh, axis=1, keepdims=True)
        mhn = mh / jnp.maximum(jnp.sqrt(mss), 1e-12)
        dmem_ref[...] = LR * jax.lax.dot_general(
            ohot, mhn, (((1,), (0,)), ((), ())),
            preferred_element_type=jnp.float32)

        ss_col = jnp.sum(ohot * s_bc, axis=1, keepdims=True)
        sstr_ref[...] = jnp.sum(jnp.broadcast_to(ss_col, (T, T)) * eye,
                                axis=0, keepdims=True)

        acc_ref[...] = jnp.zeros((QT, D), jnp.float32)
        l_ref[...] = jnp.zeros((QT, 1), jnp.float32)

    mem_blk = mem_ref[...]
    memb = mem_blk.astype(jnp.bfloat16)
    sq = mem_blk * mem_blk
    nss = jax.lax.dot_general(
        ones_row, sq, (((1,), (1,)), ((), ())),
        preferred_element_type=jnp.float32)
    rowmul = (DECAY * DECAY * str_ref[...]) / jnp.maximum(
        DECAY * jnp.sqrt(nss), 1e-12)
    logits = jax.lax.dot_general(
        qn_ref[...], memb, (((1,), (1,)), ((), ())),
        preferred_element_type=jnp.float32)
    p = jnp.exp(logits * rowmul)
    pb = p.astype(jnp.bfloat16)
    acc_ref[...] += jax.lax.dot_general(
        pb, memb, (((1,), (0,)), ((), ())),
        preferred_element_type=jnp.float32)
    l_ref[...] += jax.lax.dot_general(
        pb, jnp.ones((BM, 1), jnp.bfloat16), (((1,), (0,)), ((), ())),
        preferred_element_type=jnp.float32)

    @pl.when(j == 0)
    def _head_correction():
        qn = qn_ref[...]
        mem_head = mem_ref[0:T, :]
        dec_head = DECAY * mem_head + dmem_ref[...]
        decb = dec_head.astype(jnp.bfloat16)
        str2h = DECAY * str_ref[...][:, 0:T] + sstr_ref[...]
        nssh = jax.lax.dot_general(
            ones_row, dec_head * dec_head, (((1,), (1,)), ((), ())),
            preferred_element_type=jnp.float32)
        rmh = str2h / jnp.maximum(jnp.sqrt(nssh), 1e-12)
        lt = jax.lax.dot_general(
            qn, decb, (((1,), (1,)), ((), ())),
            preferred_element_type=jnp.float32)
        pt = jnp.exp(lt * rmh)
        ptb = pt.astype(jnp.bfloat16)
        vdecb = (dec_head * (1.0 / DECAY)).astype(jnp.bfloat16)
        memb_head = memb[0:T]
        lw = jax.lax.dot_general(
            qn, memb_head, (((1,), (1,)), ((), ())),
            preferred_element_type=jnp.float32)
        pw = jnp.exp(lw * rowmul[:, 0:T])
        pwb = pw.astype(jnp.bfloat16)
        ones_t = jnp.ones((T, 1), jnp.bfloat16)
        acc_ref[...] += (
            jax.lax.dot_general(ptb, vdecb, (((1,), (0,)), ((), ())),
                                preferred_element_type=jnp.float32)
            - jax.lax.dot_general(pwb, memb_head, (((1,), (0,)), ((), ())),
                                  preferred_element_type=jnp.float32))
        l_ref[...] += (
            jax.lax.dot_general(ptb, ones_t, (((1,), (0,)), ((), ())),
                                preferred_element_type=jnp.float32)
            - jax.lax.dot_general(pwb, ones_t, (((1,), (0,)), ((), ())),
                                  preferred_element_type=jnp.float32))

    @pl.when(j == nb - 1)
    def _finalize():
        out_ref[...] = (DECAY * acc_ref[...]) / l_ref[...]


def kernel(h, surprise, mem, strength):
    hf = h.reshape(QT, D)
    strr = strength.reshape(1, M)
    out = pl.pallas_call(
        _flash_body,
        grid=(M // BM,),
        in_specs=[
            pl.BlockSpec((QT, D), lambda j: (0, 0)),
            pl.BlockSpec((B, T), lambda j: (0, 0)),
            pl.BlockSpec((BM, D), lambda j: (j, 0)),
            pl.BlockSpec((1, BM), lambda j: (0, j)),
        ],
        out_specs=pl.BlockSpec((QT, D), lambda j: (0, 0)),
        out_shape=jax.ShapeDtypeStruct((QT, D), jnp.float32),
        scratch_shapes=[
            pltpu.VMEM((QT, D), jnp.bfloat16),
            pltpu.VMEM((QT, D), jnp.float32),
            pltpu.VMEM((QT, 1), jnp.float32),
            pltpu.VMEM((T, D), jnp.float32),
            pltpu.VMEM((1, T), jnp.float32),
        ],
        compiler_params=pltpu.CompilerParams(
            dimension_semantics=("arbitrary",)),
    )(hf, surprise, mem, strr)
    return out.reshape(B, T, D)


# speedup vs baseline: 1.3975x; 1.2278x over previous
import jax
import jax.numpy as jnp
from jax.experimental import pallas as pl
from jax.experimental.pallas import tpu as pltpu

DECAY = 0.98
LR = 0.05
B, T, D = 4, 128, 64
M = 65536
BM = 2048
QT = B * T


def _flash_body(hf_ref, sur_ref, mem_ref, str_ref, out_ref,
                qn_ref, acc_ref, dmem_ref, sstr_ref):
    j = pl.program_id(0)
    nb = pl.num_programs(0)
    ones_row = jnp.ones((1, D), jnp.float32)

    @pl.when(j == 0)
    def _prologue():
        hfv = hf_ref[...]
        qss = jnp.sum(hfv * hfv, axis=1, keepdims=True)
        qn_ref[...] = (hfv / jnp.maximum(jnp.sqrt(qss), 1e-12)
                       ).astype(jnp.bfloat16)

        s2 = jnp.mean(sur_ref[...], axis=0, keepdims=True)
        r_io = jax.lax.broadcasted_iota(jnp.int32, (T, T), 0)
        c_io = jax.lax.broadcasted_iota(jnp.int32, (T, T), 1)
        eye = (r_io == c_io).astype(jnp.float32)
        s_bc = jnp.broadcast_to(s2, (T, T))
        s_col = jnp.sum(s_bc * eye, axis=1, keepdims=True)
        gt = (s_col > s2).astype(jnp.int32)
        tie = ((s_col == s2) & (r_io < c_io)).astype(jnp.int32)
        rank = jnp.sum(gt + tie, axis=0, keepdims=True)
        ohot = (jnp.broadcast_to(rank, (T, T)) == r_io).astype(jnp.float32)

        mh = (hfv[0:T] + hfv[T:2 * T] + hfv[2 * T:3 * T] + hfv[3 * T:4 * T]) * 0.25
        mss = jnp.sum(mh * mh, axis=1, keepdims=True)
        mhn = mh / jnp.maximum(jnp.sqrt(mss), 1e-12)
        dmem_ref[...] = LR * jax.lax.dot_general(
            ohot, mhn, (((1,), (0,)), ((), ())),
            preferred_element_type=jnp.float32)

        ss_col = jnp.sum(ohot * s_bc, axis=1, keepdims=True)
        sstr_ref[...] = jnp.sum(jnp.broadcast_to(ss_col, (T, T)) * eye,
                                axis=0, keepdims=True)

        acc_ref[...] = jnp.zeros((QT, D + 1), jnp.float32)

    mem_blk = mem_ref[...]
    memb = mem_blk.astype(jnp.bfloat16)
    sq = mem_blk * mem_blk
    nss = jax.lax.dot_general(
        ones_row, sq, (((1,), (1,)), ((), ())),
        preferred_element_type=jnp.float32)
    rowmul = (DECAY * DECAY * str_ref[...]) / jnp.maximum(
        DECAY * jnp.sqrt(nss), 1e-12)
    logits = jax.lax.dot_general(
        qn_ref[...], memb, (((1,), (1,)), ((), ())),
        preferred_element_type=jnp.float32)
    p = jnp.exp(logits * rowmul)
    pb = p.astype(jnp.bfloat16)
    vaug = jnp.concatenate(
        [memb, jnp.ones((BM, 1), jnp.bfloat16)], axis=1)
    acc_ref[...] += jax.lax.dot_general(
        pb, vaug, (((1,), (0,)), ((), ())),
        preferred_element_type=jnp.float32)

    @pl.when(j == 0)
    def _head_correction():
        qn = qn_ref[...]
        mem_head = mem_ref[0:T, :]
        dec_head = DECAY * mem_head + dmem_ref[...]
        decb = dec_head.astype(jnp.bfloat16)
        str2h = DECAY * str_ref[...][:, 0:T] + sstr_ref[...]
        nssh = jax.lax.dot_general(
            ones_row, dec_head * dec_head, (((1,), (1,)), ((), ())),
            preferred_element_type=jnp.float32)
        rmh = str2h / jnp.maximum(jnp.sqrt(nssh), 1e-12)
        lt = jax.lax.dot_general(
            qn, decb, (((1,), (1,)), ((), ())),
            preferred_element_type=jnp.float32)
        pt = jnp.exp(lt * rmh)
        ptb = pt.astype(jnp.bfloat16)
        vdecb = (dec_head * (1.0 / DECAY)).astype(jnp.bfloat16)
        memb_head = memb[0:T]
        lw = jax.lax.dot_general(
            qn, memb_head, (((1,), (1,)), ((), ())),
            preferred_element_type=jnp.float32)
        pw = jnp.exp(lw * rowmul[:, 0:T])
        pwb = pw.astype(jnp.bfloat16)
        vdec_aug = jnp.concatenate(
            [vdecb, jnp.ones((T, 1), jnp.bfloat16)], axis=1)
        mem_head_aug = jnp.concatenate(
            [memb_head, jnp.ones((T, 1), jnp.bfloat16)], axis=1)
        acc_ref[...] += (
            jax.lax.dot_general(ptb, vdec_aug, (((1,), (0,)), ((), ())),
                                preferred_element_type=jnp.float32)
            - jax.lax.dot_general(pwb, mem_head_aug, (((1,), (0,)), ((), ())),
                                  preferred_element_type=jnp.float32))

    @pl.when(j == nb - 1)
    def _finalize():
        acc = acc_ref[...]
        out_ref[...] = (DECAY * acc[:, :D]) / acc[:, D:D + 1]


def kernel(h, surprise, mem, strength):
    hf = h.reshape(QT, D)
    strr = strength.reshape(1, M)
    out = pl.pallas_call(
        _flash_body,
        grid=(M // BM,),
        in_specs=[
            pl.BlockSpec((QT, D), lambda j: (0, 0)),
            pl.BlockSpec((B, T), lambda j: (0, 0)),
            pl.BlockSpec((BM, D), lambda j: (j, 0)),
            pl.BlockSpec((1, BM), lambda j: (0, j)),
        ],
        out_specs=pl.BlockSpec((QT, D), lambda j: (0, 0)),
        out_shape=jax.ShapeDtypeStruct((QT, D), jnp.float32),
        scratch_shapes=[
            pltpu.VMEM((QT, D), jnp.bfloat16),
            pltpu.VMEM((QT, D + 1), jnp.float32),
            pltpu.VMEM((T, D), jnp.float32),
            pltpu.VMEM((1, T), jnp.float32),
        ],
        compiler_params=pltpu.CompilerParams(
            dimension_semantics=("arbitrary",)),
    )(hf, surprise, mem, strr)
    return out.reshape(B, T, D)
